# baseline (speedup 1.0000x reference)
import jax
import jax.numpy as jnp
import numpy as np
from jax import lax
from jax.experimental import pallas as pl
from jax.experimental.pallas import tpu as pltpu
from jax.experimental.pallas import tpu_sc as plsc

N = 32768
F = 128
Z = 128
K = 256
TILES = 32
TILE_ROWS = N // TILES
NSC = 16
CHUNK = N // NSC
NV = CHUNK // 16
MININT = np.int32(-2147483648)


def _scores_body(bags_ref, wenc_ref, benc_ref, watt_ref, out_ref):
    emb = jnp.maximum(
        jnp.dot(bags_ref[...], wenc_ref[...], preferred_element_type=jnp.float32)
        + benc_ref[...],
        0.0,
    )
    s = lax.dot_general(
        watt_ref[...], emb, (((1,), (1,)), ((), ())),
        preferred_element_type=jnp.float32,
    )
    out_ref[...] = s.reshape(1, 1, TILE_ROWS)


_scores_call = pl.pallas_call(
    _scores_body,
    grid=(TILES,),
    in_specs=[
        pl.BlockSpec((TILE_ROWS, F), lambda i: (i, 0)),
        pl.BlockSpec((F, Z), lambda i: (0, 0)),
        pl.BlockSpec((1, Z), lambda i: (0, 0)),
        pl.BlockSpec((1, Z), lambda i: (0, 0)),
    ],
    out_specs=pl.BlockSpec((1, 1, TILE_ROWS), lambda i: (i, 0, 0)),
    out_shape=jax.ShapeDtypeStruct((TILES, 1, TILE_ROWS), jnp.float32),
)


def _iota16():
    return lax.iota(jnp.int32, 16)


def _sc_body(scores_hbm, bags_hbm, rows_hbm, shist, scnt, sidx_hbm,
             kbuf, hist, allh, cbuf, allcnt, selbuf, sidxall, gidx16,
             rowbuf16, sem, sem2):
    wid = lax.axis_index("s")
    base = wid * CHUNK
    iota = _iota16()
    ones = jnp.ones((16,), jnp.int32)

    pltpu.sync_copy(scores_hbm.at[pl.ds(base, CHUNK)], kbuf)

    def key_body(i, c):
        ib = kbuf[pl.ds(i * 16, 16)]
        kbuf[pl.ds(i * 16, 16)] = jnp.where(ib < 0, ~ib, ib | MININT)
        return c

    lax.fori_loop(0, NV, key_body, 0)

    hi = jnp.int32(0)
    k_rem = jnp.int32(K)
    for shift in (24, 16, 8, 0):
        um = (0xFFFFFFFF << (shift + 8)) & 0xFFFFFFFF
        upper_mask = jnp.int32(um - 0x100000000 if um >= 0x80000000 else um)
        for v in range(16):
            hist[pl.ds(v * 16, 16)] = jnp.zeros((16,), jnp.int32)
        hi_upper = hi & upper_mask

        def hist_body(i, c, shift=shift, upper_mask=upper_mask, hi_upper=hi_upper):
            kb = kbuf[pl.ds(i * 16, 16)]
            byte = lax.shift_right_logical(kb, shift) & 0xFF
            match = (kb & upper_mask) == hi_upper
            plsc.addupdate_scatter(hist, [byte], ones, mask=match)
            return c

        lax.fori_loop(0, NV, hist_body, 0)

        pltpu.sync_copy(hist, shist.at[wid])
        plsc.subcore_barrier()
        pltpu.sync_copy(shist, allh)

        tot = []
        for v in range(16):
            tv = jnp.zeros((16,), jnp.int32)
            for t in range(16):
                tv = tv + allh[t, pl.ds(v * 16, 16)]
            tot.append(tv)
        desc_incl = [None] * 16
        carry = jnp.int32(0)
        for v in range(15, -1, -1):
            rv = lax.rev(tot[v], (0,))
            cs = plsc.cumsum(rv) + carry
            desc_incl[v] = lax.rev(cs, (0,))
            carry = carry + jnp.sum(tot[v])
        b_star = jnp.int32(0)
        new_krem = jnp.int32(0)
        for v in range(16):
            cgt = desc_incl[v] - tot[v]
            cond = (cgt < k_rem) & (desc_incl[v] >= k_rem)
            ids = iota + 16 * v
            b_star = b_star + jnp.sum(jnp.where(cond, ids, 0))
            new_krem = new_krem + jnp.sum(jnp.where(cond, k_rem - cgt, 0))
        k_rem = new_krem
        hi = hi | (b_star << shift)
        plsc.subcore_barrier()

    vk_bits = hi
    vk_s = vk_bits ^ MININT
    need = k_rem

    def cnt_body(i, c):
        cgt, ceq = c
        kb = kbuf[pl.ds(i * 16, 16)]
        sk = kb ^ MININT
        cgt = cgt + jnp.sum(jnp.where(sk > vk_s, 1, 0))
        ceq = ceq + jnp.sum(jnp.where(kb == vk_bits, 1, 0))
        return (cgt, ceq)

    cgt_me, ceq_me = lax.fori_loop(0, NV, cnt_body, (jnp.int32(0), jnp.int32(0)))

    cbuf[...] = jnp.where(iota == 0, cgt_me, jnp.where(iota == 1, ceq_me, 0))
    pltpu.sync_copy(cbuf, scnt.at[wid])
    plsc.subcore_barrier()
    pltpu.sync_copy(scnt, allcnt)

    my_pre_eq = jnp.int32(0)
    pre_eq = jnp.int32(0)
    pre_sel = jnp.int32(0)
    bases = []
    sizes = []
    for t in range(16):
        row = allcnt[t, :]
        g_t = jnp.sum(jnp.where(iota == 0, row, 0))
        e_t = jnp.sum(jnp.where(iota == 1, row, 0))
        take_t = jnp.clip(need - pre_eq, 0, e_t)
        is_me = jnp.int32(t) == wid
        my_pre_eq = jnp.where(is_me, pre_eq, my_pre_eq)
        bases.append(pre_sel)
        sizes.append(g_t + take_t)
        pre_eq = pre_eq + e_t
        pre_sel = pre_sel + g_t + take_t
    my_take = jnp.clip(need - my_pre_eq, 0, ceq_me)
    c_me = cgt_me + my_take

    for v in range(16):
        selbuf[pl.ds(v * 16, 16)] = jnp.zeros((16,), jnp.int32)

    def sel_body(i, c):
        sel_c, eq_c = c
        kb = kbuf[pl.ds(i * 16, 16)]
        sk = kb ^ MININT
        gtm = sk > vk_s
        eqm = kb == vk_bits
        eq_rank = plsc.cumsum(jnp.where(eqm, 1, 0)) - 1 + eq_c
        selm = gtm | (eqm & ((my_pre_eq + eq_rank) < need))
        seli = jnp.where(selm, 1, 0)
        pos = plsc.cumsum(seli) - 1 + sel_c
        gidx = base + i * 16 + iota
        plsc.store_scatter(selbuf, [pos], gidx, mask=selm)
        return (sel_c + jnp.sum(seli), eq_c + jnp.sum(jnp.where(eqm, 1, 0)))

    lax.fori_loop(0, NV, sel_body, (jnp.int32(0), jnp.int32(0)))

    pltpu.sync_copy(selbuf, sidx_hbm.at[wid])
    plsc.subcore_barrier()
    pltpu.sync_copy(sidx_hbm, sidxall)

    r = wid * 16 + iota
    src_t = jnp.zeros((16,), jnp.int32)
    src_o = jnp.zeros((16,), jnp.int32)
    for t in range(16):
        m = (r >= bases[t]) & (r < bases[t] + sizes[t])
        src_t = src_t + jnp.where(m, t, 0)
        src_o = src_o + jnp.where(m, r - bases[t], 0)
    gidx16[...] = plsc.load_gather(sidxall, [src_t, src_o])
    pltpu.async_copy(bags_hbm.at[gidx16], rowbuf16, sem).wait()
    pltpu.sync_copy(rowbuf16, rows_hbm.at[pl.ds(wid * 16, 16)])


_sc_mesh = plsc.VectorSubcoreMesh(
    core_axis_name="c", subcore_axis_name="s", num_cores=1, num_subcores=NSC)


def _make_sc_call(interpret=False):
    return pl.kernel(
        _sc_body,
        out_type=[
            jax.ShapeDtypeStruct((K, F), jnp.float32),
            jax.ShapeDtypeStruct((16, 256), jnp.int32),
            jax.ShapeDtypeStruct((16, 16), jnp.int32),
            jax.ShapeDtypeStruct((16, K), jnp.int32),
        ],
        mesh=_sc_mesh,
        scratch_types=[
            pltpu.VMEM((CHUNK,), jnp.int32),
            pltpu.VMEM((256,), jnp.int32),
            pltpu.VMEM((16, 256), jnp.int32),
            pltpu.VMEM((16,), jnp.int32),
            pltpu.VMEM((16, 16), jnp.int32),
            pltpu.VMEM((K,), jnp.int32),
            pltpu.VMEM((16, K), jnp.int32),
            pltpu.VMEM((16,), jnp.int32),
            pltpu.VMEM((16, F), jnp.float32),
            pltpu.SemaphoreType.DMA,
            pltpu.SemaphoreType.DMA,
        ],
        compiler_params=pltpu.CompilerParams(needs_layout_passes=False),
        interpret=interpret,
    )


_sc_call = _make_sc_call()


def _head_body(rows_ref, wenc_ref, benc_ref, gamma_ref, beta_ref,
               mean_ref, var_ref, whead_ref, bhead_ref, out_ref):
    emb = jnp.maximum(
        jnp.dot(rows_ref[...], wenc_ref[...], preferred_element_type=jnp.float32)
        + benc_ref[...],
        0.0,
    )
    pooled = jnp.sum(emb, axis=0, keepdims=True) * (1.0 / K)
    h = (pooled - mean_ref[...]) * lax.rsqrt(var_ref[...] + 1e-5) * gamma_ref[...] + beta_ref[...]
    out_ref[...] = jnp.dot(h, whead_ref[...], preferred_element_type=jnp.float32) + bhead_ref[...]


_head_call = pl.pallas_call(
    _head_body,
    in_specs=[
        pl.BlockSpec((K, F), lambda: (0, 0)),
        pl.BlockSpec((F, Z), lambda: (0, 0)),
        pl.BlockSpec((1, Z), lambda: (0, 0)),
        pl.BlockSpec((1, Z), lambda: (0, 0)),
        pl.BlockSpec((1, Z), lambda: (0, 0)),
        pl.BlockSpec((1, Z), lambda: (0, 0)),
        pl.BlockSpec((1, Z), lambda: (0, 0)),
        pl.BlockSpec((Z, 2), lambda: (0, 0)),
        pl.BlockSpec((1, 2), lambda: (0, 0)),
    ],
    out_specs=pl.BlockSpec((1, 2), lambda: (0, 0)),
    out_shape=jax.ShapeDtypeStruct((1, 2), jnp.float32),
)


def kernel(bags, W_enc, b_enc, W_att, b_att, bn_gamma, bn_beta, bn_mean,
           bn_var, W_head, b_head):
    benc2 = b_enc.reshape(1, Z)
    watt2 = W_att.reshape(1, Z)
    scores = _scores_call(bags, W_enc, benc2, watt2).reshape(N)
    scores_i32 = lax.bitcast_convert_type(scores, jnp.int32)
    rows = _sc_call(scores_i32, bags)[0]
    out = _head_call(
        rows, W_enc, benc2,
        bn_gamma.reshape(1, Z), bn_beta.reshape(1, Z),
        bn_mean.reshape(1, Z), bn_var.reshape(1, Z),
        W_head, b_head.reshape(1, 2),
    )
    return out.reshape(2)


# speedup vs baseline: 1.2587x; 1.2587x over previous
import jax
import jax.numpy as jnp
import numpy as np
from jax import lax
from jax.experimental import pallas as pl
from jax.experimental.pallas import tpu as pltpu
from jax.experimental.pallas import tpu_sc as plsc

N = 32768
F = 128
Z = 128
K = 256
TILES = 8
TILE_ROWS = N // TILES
NSC = 16
CHUNK = N // NSC
NV = CHUNK // 16
MININT = np.int32(-2147483648)


def _scores_body(bags_ref, wenc_ref, benc_ref, watt_ref, out_ref):
    emb = jnp.maximum(
        jnp.dot(bags_ref[...], wenc_ref[...], preferred_element_type=jnp.float32)
        + benc_ref[...],
        0.0,
    )
    s = lax.dot_general(
        watt_ref[...], emb, (((1,), (1,)), ((), ())),
        preferred_element_type=jnp.float32,
    )
    out_ref[...] = s.reshape(1, 1, TILE_ROWS)


_scores_call = pl.pallas_call(
    _scores_body,
    grid=(TILES,),
    in_specs=[
        pl.BlockSpec((TILE_ROWS, F), lambda i: (i, 0)),
        pl.BlockSpec((F, Z), lambda i: (0, 0)),
        pl.BlockSpec((1, Z), lambda i: (0, 0)),
        pl.BlockSpec((1, Z), lambda i: (0, 0)),
    ],
    out_specs=pl.BlockSpec((1, 1, TILE_ROWS), lambda i: (i, 0, 0)),
    out_shape=jax.ShapeDtypeStruct((TILES, 1, TILE_ROWS), jnp.float32),
)


def _iota16():
    return lax.iota(jnp.int32, 16)


def _sc_body(scores_hbm, bags_hbm, rows_hbm, shist, scnt, sidx_hbm,
             kbuf, hist, allh, cbuf, allcnt, selbuf, sidxall, gidx16,
             rowbuf16, sem, sem2):
    wid = lax.axis_index("s")
    base = wid * CHUNK
    iota = _iota16()
    ones = jnp.ones((16,), jnp.int32)

    pltpu.sync_copy(scores_hbm.at[pl.ds(base, CHUNK)], kbuf)

    def key_body(i, c):
        ib = kbuf[pl.ds(i * 16, 16)]
        kbuf[pl.ds(i * 16, 16)] = jnp.where(ib < 0, ~ib, ib | MININT)
        return c

    lax.fori_loop(0, NV, key_body, 0)

    hi = jnp.int32(0)
    k_rem = jnp.int32(K)
    for shift in (24, 16, 8, 0):
        um = (0xFFFFFFFF << (shift + 8)) & 0xFFFFFFFF
        upper_mask = jnp.int32(um - 0x100000000 if um >= 0x80000000 else um)
        for v in range(16):
            hist[pl.ds(v * 16, 16)] = jnp.zeros((16,), jnp.int32)
        hi_upper = hi & upper_mask

        def hist_body(i, c, shift=shift, upper_mask=upper_mask, hi_upper=hi_upper):
            kb = kbuf[pl.ds(i * 16, 16)]
            byte = lax.shift_right_logical(kb, shift) & 0xFF
            match = (kb & upper_mask) == hi_upper
            plsc.addupdate_scatter(hist, [byte], ones, mask=match)
            return c

        lax.fori_loop(0, NV, hist_body, 0)

        pltpu.sync_copy(hist, shist.at[wid])
        plsc.subcore_barrier()
        pltpu.sync_copy(shist, allh)

        tot = []
        for v in range(16):
            tv = jnp.zeros((16,), jnp.int32)
            for t in range(16):
                tv = tv + allh[t, pl.ds(v * 16, 16)]
            tot.append(tv)
        desc_incl = [None] * 16
        carry = jnp.int32(0)
        for v in range(15, -1, -1):
            rv = lax.rev(tot[v], (0,))
            cs = plsc.cumsum(rv) + carry
            desc_incl[v] = lax.rev(cs, (0,))
            carry = carry + jnp.sum(tot[v])
        b_star = jnp.int32(0)
        new_krem = jnp.int32(0)
        for v in range(16):
            cgt = desc_incl[v] - tot[v]
            cond = (cgt < k_rem) & (desc_incl[v] >= k_rem)
            ids = iota + 16 * v
            b_star = b_star + jnp.sum(jnp.where(cond, ids, 0))
            new_krem = new_krem + jnp.sum(jnp.where(cond, k_rem - cgt, 0))
        k_rem = new_krem
        hi = hi | (b_star << shift)
        plsc.subcore_barrier()

    vk_bits = hi
    vk_s = vk_bits ^ MININT
    need = k_rem

    def cnt_body(i, c):
        cgt, ceq = c
        kb = kbuf[pl.ds(i * 16, 16)]
        sk = kb ^ MININT
        cgt = cgt + jnp.sum(jnp.where(sk > vk_s, 1, 0))
        ceq = ceq + jnp.sum(jnp.where(kb == vk_bits, 1, 0))
        return (cgt, ceq)

    cgt_me, ceq_me = lax.fori_loop(0, NV, cnt_body, (jnp.int32(0), jnp.int32(0)))

    cbuf[...] = jnp.where(iota == 0, cgt_me, jnp.where(iota == 1, ceq_me, 0))
    pltpu.sync_copy(cbuf, scnt.at[wid])
    plsc.subcore_barrier()
    pltpu.sync_copy(scnt, allcnt)

    my_pre_eq = jnp.int32(0)
    pre_eq = jnp.int32(0)
    pre_sel = jnp.int32(0)
    bases = []
    sizes = []
    for t in range(16):
        row = allcnt[t, :]
        g_t = jnp.sum(jnp.where(iota == 0, row, 0))
        e_t = jnp.sum(jnp.where(iota == 1, row, 0))
        take_t = jnp.clip(need - pre_eq, 0, e_t)
        is_me = jnp.int32(t) == wid
        my_pre_eq = jnp.where(is_me, pre_eq, my_pre_eq)
        bases.append(pre_sel)
        sizes.append(g_t + take_t)
        pre_eq = pre_eq + e_t
        pre_sel = pre_sel + g_t + take_t
    my_take = jnp.clip(need - my_pre_eq, 0, ceq_me)
    c_me = cgt_me + my_take

    for v in range(16):
        selbuf[pl.ds(v * 16, 16)] = jnp.zeros((16,), jnp.int32)

    def sel_body(i, c):
        sel_c, eq_c = c
        kb = kbuf[pl.ds(i * 16, 16)]
        sk = kb ^ MININT
        gtm = sk > vk_s
        eqm = kb == vk_bits
        eq_rank = plsc.cumsum(jnp.where(eqm, 1, 0)) - 1 + eq_c
        selm = gtm | (eqm & ((my_pre_eq + eq_rank) < need))
        seli = jnp.where(selm, 1, 0)
        pos = plsc.cumsum(seli) - 1 + sel_c
        gidx = base + i * 16 + iota
        plsc.store_scatter(selbuf, [pos], gidx, mask=selm)
        return (sel_c + jnp.sum(seli), eq_c + jnp.sum(jnp.where(eqm, 1, 0)))

    lax.fori_loop(0, NV, sel_body, (jnp.int32(0), jnp.int32(0)))

    pltpu.sync_copy(selbuf, sidx_hbm.at[wid])
    plsc.subcore_barrier()
    pltpu.sync_copy(sidx_hbm, sidxall)

    r = wid * 16 + iota
    src_t = jnp.zeros((16,), jnp.int32)
    src_o = jnp.zeros((16,), jnp.int32)
    for t in range(16):
        m = (r >= bases[t]) & (r < bases[t] + sizes[t])
        src_t = src_t + jnp.where(m, t, 0)
        src_o = src_o + jnp.where(m, r - bases[t], 0)
    gidx16[...] = plsc.load_gather(sidxall, [src_t, src_o])
    pltpu.async_copy(bags_hbm.at[gidx16], rowbuf16, sem).wait()
    pltpu.sync_copy(rowbuf16, rows_hbm.at[pl.ds(wid * 16, 16)])


_sc_mesh = plsc.VectorSubcoreMesh(
    core_axis_name="c", subcore_axis_name="s", num_cores=1, num_subcores=NSC)


def _make_sc_call(interpret=False):
    return pl.kernel(
        _sc_body,
        out_type=[
            jax.ShapeDtypeStruct((K, F), jnp.float32),
            jax.ShapeDtypeStruct((16, 256), jnp.int32),
            jax.ShapeDtypeStruct((16, 16), jnp.int32),
            jax.ShapeDtypeStruct((16, K), jnp.int32),
        ],
        mesh=_sc_mesh,
        scratch_types=[
            pltpu.VMEM((CHUNK,), jnp.int32),
            pltpu.VMEM((256,), jnp.int32),
            pltpu.VMEM((16, 256), jnp.int32),
            pltpu.VMEM((16,), jnp.int32),
            pltpu.VMEM((16, 16), jnp.int32),
            pltpu.VMEM((K,), jnp.int32),
            pltpu.VMEM((16, K), jnp.int32),
            pltpu.VMEM((16,), jnp.int32),
            pltpu.VMEM((16, F), jnp.float32),
            pltpu.SemaphoreType.DMA,
            pltpu.SemaphoreType.DMA,
        ],
        compiler_params=pltpu.CompilerParams(needs_layout_passes=False),
        interpret=interpret,
    )


_sc_call = _make_sc_call()


def _head_body(rows_ref, wenc_ref, benc_ref, gamma_ref, beta_ref,
               mean_ref, var_ref, whead_ref, bhead_ref, out_ref):
    emb = jnp.maximum(
        jnp.dot(rows_ref[...], wenc_ref[...], preferred_element_type=jnp.float32)
        + benc_ref[...],
        0.0,
    )
    pooled = jnp.sum(emb, axis=0, keepdims=True) * (1.0 / K)
    h = (pooled - mean_ref[...]) * lax.rsqrt(var_ref[...] + 1e-5) * gamma_ref[...] + beta_ref[...]
    out_ref[...] = jnp.dot(h, whead_ref[...], preferred_element_type=jnp.float32) + bhead_ref[...]


_head_call = pl.pallas_call(
    _head_body,
    in_specs=[
        pl.BlockSpec((K, F), lambda: (0, 0)),
        pl.BlockSpec((F, Z), lambda: (0, 0)),
        pl.BlockSpec((1, Z), lambda: (0, 0)),
        pl.BlockSpec((1, Z), lambda: (0, 0)),
        pl.BlockSpec((1, Z), lambda: (0, 0)),
        pl.BlockSpec((1, Z), lambda: (0, 0)),
        pl.BlockSpec((1, Z), lambda: (0, 0)),
        pl.BlockSpec((Z, 2), lambda: (0, 0)),
        pl.BlockSpec((1, 2), lambda: (0, 0)),
    ],
    out_specs=pl.BlockSpec((1, 2), lambda: (0, 0)),
    out_shape=jax.ShapeDtypeStruct((1, 2), jnp.float32),
)


def kernel(bags, W_enc, b_enc, W_att, b_att, bn_gamma, bn_beta, bn_mean,
           bn_var, W_head, b_head):
    benc2 = b_enc.reshape(1, Z)
    watt2 = W_att.reshape(1, Z)
    scores = _scores_call(bags, W_enc, benc2, watt2).reshape(N)
    scores_i32 = lax.bitcast_convert_type(scores, jnp.int32)
    rows = _sc_call(scores_i32, bags)[0]
    out = _head_call(
        rows, W_enc, benc2,
        bn_gamma.reshape(1, Z), bn_beta.reshape(1, Z),
        bn_mean.reshape(1, Z), bn_var.reshape(1, Z),
        W_head, b_head.reshape(1, 2),
    )
    return out.reshape(2)


# speedup vs baseline: 1.3089x; 1.0398x over previous
import jax
import jax.numpy as jnp
import numpy as np
from jax import lax
from jax.experimental import pallas as pl
from jax.experimental.pallas import tpu as pltpu
from jax.experimental.pallas import tpu_sc as plsc

N = 32768
F = 128
Z = 128
K = 256
TILES = 4
TILE_ROWS = N // TILES
NSC = 16
CHUNK = N // NSC
NV = CHUNK // 16
MININT = np.int32(-2147483648)


def _scores_body(bags_ref, wenc_ref, benc_ref, watt_ref, out_ref):
    emb = jnp.maximum(
        jnp.dot(bags_ref[...], wenc_ref[...], preferred_element_type=jnp.float32)
        + benc_ref[...],
        0.0,
    )
    s = lax.dot_general(
        watt_ref[...], emb, (((1,), (1,)), ((), ())),
        preferred_element_type=jnp.float32,
    )
    out_ref[...] = s.reshape(1, 1, TILE_ROWS)


_scores_call = pl.pallas_call(
    _scores_body,
    grid=(TILES,),
    in_specs=[
        pl.BlockSpec((TILE_ROWS, F), lambda i: (i, 0)),
        pl.BlockSpec((F, Z), lambda i: (0, 0)),
        pl.BlockSpec((1, Z), lambda i: (0, 0)),
        pl.BlockSpec((1, Z), lambda i: (0, 0)),
    ],
    out_specs=pl.BlockSpec((1, 1, TILE_ROWS), lambda i: (i, 0, 0)),
    out_shape=jax.ShapeDtypeStruct((TILES, 1, TILE_ROWS), jnp.float32),
)


def _iota16():
    return lax.iota(jnp.int32, 16)


def _sc_body(scores_hbm, bags_hbm, rows_hbm, shist, scnt, sidx_hbm,
             kbuf, hist, allh, cbuf, allcnt, selbuf, sidxall, gidx16,
             rowbuf16, sem, sem2):
    wid = lax.axis_index("s")
    base = wid * CHUNK
    iota = _iota16()
    ones = jnp.ones((16,), jnp.int32)

    pltpu.sync_copy(scores_hbm.at[pl.ds(base, CHUNK)], kbuf)

    def key_body(i, c):
        ib = kbuf[pl.ds(i * 16, 16)]
        kbuf[pl.ds(i * 16, 16)] = jnp.where(ib < 0, ~ib, ib | MININT)
        return c

    lax.fori_loop(0, NV, key_body, 0)

    hi = jnp.int32(0)
    k_rem = jnp.int32(K)
    for shift in (24, 16, 8, 0):
        um = (0xFFFFFFFF << (shift + 8)) & 0xFFFFFFFF
        upper_mask = jnp.int32(um - 0x100000000 if um >= 0x80000000 else um)
        for v in range(16):
            hist[pl.ds(v * 16, 16)] = jnp.zeros((16,), jnp.int32)
        hi_upper = hi & upper_mask

        def hist_body(i, c, shift=shift, upper_mask=upper_mask, hi_upper=hi_upper):
            kb = kbuf[pl.ds(i * 16, 16)]
            byte = lax.shift_right_logical(kb, shift) & 0xFF
            match = (kb & upper_mask) == hi_upper
            plsc.addupdate_scatter(hist, [byte], ones, mask=match)
            return c

        lax.fori_loop(0, NV, hist_body, 0)

        pltpu.sync_copy(hist, shist.at[wid])
        plsc.subcore_barrier()
        pltpu.sync_copy(shist, allh)

        tot = []
        for v in range(16):
            tv = jnp.zeros((16,), jnp.int32)
            for t in range(16):
                tv = tv + allh[t, pl.ds(v * 16, 16)]
            tot.append(tv)
        desc_incl = [None] * 16
        carry = jnp.int32(0)
        for v in range(15, -1, -1):
            rv = lax.rev(tot[v], (0,))
            cs = plsc.cumsum(rv) + carry
            desc_incl[v] = lax.rev(cs, (0,))
            carry = carry + jnp.sum(tot[v])
        b_star = jnp.int32(0)
        new_krem = jnp.int32(0)
        for v in range(16):
            cgt = desc_incl[v] - tot[v]
            cond = (cgt < k_rem) & (desc_incl[v] >= k_rem)
            ids = iota + 16 * v
            b_star = b_star + jnp.sum(jnp.where(cond, ids, 0))
            new_krem = new_krem + jnp.sum(jnp.where(cond, k_rem - cgt, 0))
        k_rem = new_krem
        hi = hi | (b_star << shift)
        plsc.subcore_barrier()

    vk_bits = hi
    vk_s = vk_bits ^ MININT
    need = k_rem

    def cnt_body(i, c):
        cgt, ceq = c
        kb = kbuf[pl.ds(i * 16, 16)]
        sk = kb ^ MININT
        cgt = cgt + jnp.sum(jnp.where(sk > vk_s, 1, 0))
        ceq = ceq + jnp.sum(jnp.where(kb == vk_bits, 1, 0))
        return (cgt, ceq)

    cgt_me, ceq_me = lax.fori_loop(0, NV, cnt_body, (jnp.int32(0), jnp.int32(0)))

    cbuf[...] = jnp.where(iota == 0, cgt_me, jnp.where(iota == 1, ceq_me, 0))
    pltpu.sync_copy(cbuf, scnt.at[wid])
    plsc.subcore_barrier()
    pltpu.sync_copy(scnt, allcnt)

    my_pre_eq = jnp.int32(0)
    pre_eq = jnp.int32(0)
    pre_sel = jnp.int32(0)
    bases = []
    sizes = []
    for t in range(16):
        row = allcnt[t, :]
        g_t = jnp.sum(jnp.where(iota == 0, row, 0))
        e_t = jnp.sum(jnp.where(iota == 1, row, 0))
        take_t = jnp.clip(need - pre_eq, 0, e_t)
        is_me = jnp.int32(t) == wid
        my_pre_eq = jnp.where(is_me, pre_eq, my_pre_eq)
        bases.append(pre_sel)
        sizes.append(g_t + take_t)
        pre_eq = pre_eq + e_t
        pre_sel = pre_sel + g_t + take_t
    my_take = jnp.clip(need - my_pre_eq, 0, ceq_me)
    c_me = cgt_me + my_take

    for v in range(16):
        selbuf[pl.ds(v * 16, 16)] = jnp.zeros((16,), jnp.int32)

    def sel_body(i, c):
        sel_c, eq_c = c
        kb = kbuf[pl.ds(i * 16, 16)]
        sk = kb ^ MININT
        gtm = sk > vk_s
        eqm = kb == vk_bits
        eq_rank = plsc.cumsum(jnp.where(eqm, 1, 0)) - 1 + eq_c
        selm = gtm | (eqm & ((my_pre_eq + eq_rank) < need))
        seli = jnp.where(selm, 1, 0)
        pos = plsc.cumsum(seli) - 1 + sel_c
        gidx = base + i * 16 + iota
        plsc.store_scatter(selbuf, [pos], gidx, mask=selm)
        return (sel_c + jnp.sum(seli), eq_c + jnp.sum(jnp.where(eqm, 1, 0)))

    lax.fori_loop(0, NV, sel_body, (jnp.int32(0), jnp.int32(0)))

    pltpu.sync_copy(selbuf, sidx_hbm.at[wid])
    plsc.subcore_barrier()
    pltpu.sync_copy(sidx_hbm, sidxall)

    r = wid * 16 + iota
    src_t = jnp.zeros((16,), jnp.int32)
    src_o = jnp.zeros((16,), jnp.int32)
    for t in range(16):
        m = (r >= bases[t]) & (r < bases[t] + sizes[t])
        src_t = src_t + jnp.where(m, t, 0)
        src_o = src_o + jnp.where(m, r - bases[t], 0)
    gidx16[...] = plsc.load_gather(sidxall, [src_t, src_o])
    pltpu.async_copy(bags_hbm.at[gidx16], rowbuf16, sem).wait()
    pltpu.sync_copy(rowbuf16, rows_hbm.at[pl.ds(wid * 16, 16)])


_sc_mesh = plsc.VectorSubcoreMesh(
    core_axis_name="c", subcore_axis_name="s", num_cores=1, num_subcores=NSC)


def _make_sc_call(interpret=False):
    return pl.kernel(
        _sc_body,
        out_type=[
            jax.ShapeDtypeStruct((K, F), jnp.float32),
            jax.ShapeDtypeStruct((16, 256), jnp.int32),
            jax.ShapeDtypeStruct((16, 16), jnp.int32),
            jax.ShapeDtypeStruct((16, K), jnp.int32),
        ],
        mesh=_sc_mesh,
        scratch_types=[
            pltpu.VMEM((CHUNK,), jnp.int32),
            pltpu.VMEM((256,), jnp.int32),
            pltpu.VMEM((16, 256), jnp.int32),
            pltpu.VMEM((16,), jnp.int32),
            pltpu.VMEM((16, 16), jnp.int32),
            pltpu.VMEM((K,), jnp.int32),
            pltpu.VMEM((16, K), jnp.int32),
            pltpu.VMEM((16,), jnp.int32),
            pltpu.VMEM((16, F), jnp.float32),
            pltpu.SemaphoreType.DMA,
            pltpu.SemaphoreType.DMA,
        ],
        compiler_params=pltpu.CompilerParams(needs_layout_passes=False),
        interpret=interpret,
    )


_sc_call = _make_sc_call()


def _head_body(rows_ref, wenc_ref, benc_ref, gamma_ref, beta_ref,
               mean_ref, var_ref, whead_ref, bhead_ref, out_ref):
    emb = jnp.maximum(
        jnp.dot(rows_ref[...], wenc_ref[...], preferred_element_type=jnp.float32)
        + benc_ref[...],
        0.0,
    )
    pooled = jnp.sum(emb, axis=0, keepdims=True) * (1.0 / K)
    h = (pooled - mean_ref[...]) * lax.rsqrt(var_ref[...] + 1e-5) * gamma_ref[...] + beta_ref[...]
    out_ref[...] = jnp.dot(h, whead_ref[...], preferred_element_type=jnp.float32) + bhead_ref[...]


_head_call = pl.pallas_call(
    _head_body,
    in_specs=[
        pl.BlockSpec((K, F), lambda: (0, 0)),
        pl.BlockSpec((F, Z), lambda: (0, 0)),
        pl.BlockSpec((1, Z), lambda: (0, 0)),
        pl.BlockSpec((1, Z), lambda: (0, 0)),
        pl.BlockSpec((1, Z), lambda: (0, 0)),
        pl.BlockSpec((1, Z), lambda: (0, 0)),
        pl.BlockSpec((1, Z), lambda: (0, 0)),
        pl.BlockSpec((Z, 2), lambda: (0, 0)),
        pl.BlockSpec((1, 2), lambda: (0, 0)),
    ],
    out_specs=pl.BlockSpec((1, 2), lambda: (0, 0)),
    out_shape=jax.ShapeDtypeStruct((1, 2), jnp.float32),
)


def kernel(bags, W_enc, b_enc, W_att, b_att, bn_gamma, bn_beta, bn_mean,
           bn_var, W_head, b_head):
    benc2 = b_enc.reshape(1, Z)
    watt2 = W_att.reshape(1, Z)
    scores = _scores_call(bags, W_enc, benc2, watt2).reshape(N)
    scores_i32 = lax.bitcast_convert_type(scores, jnp.int32)
    rows = _sc_call(scores_i32, bags)[0]
    out = _head_call(
        rows, W_enc, benc2,
        bn_gamma.reshape(1, Z), bn_beta.reshape(1, Z),
        bn_mean.reshape(1, Z), bn_var.reshape(1, Z),
        W_head, b_head.reshape(1, 2),
    )
    return out.reshape(2)


# speedup vs baseline: 1.3130x; 1.0032x over previous
import jax
import jax.numpy as jnp
import numpy as np
from jax import lax
from jax.experimental import pallas as pl
from jax.experimental.pallas import tpu as pltpu
from jax.experimental.pallas import tpu_sc as plsc

N = 32768
F = 128
Z = 128
K = 256
TILES = 2
TILE_ROWS = N // TILES
NSC = 16
CHUNK = N // NSC
NV = CHUNK // 16
MININT = np.int32(-2147483648)


def _scores_body(bags_ref, wenc_ref, benc_ref, watt_ref, out_ref):
    emb = jnp.maximum(
        jnp.dot(bags_ref[...], wenc_ref[...], preferred_element_type=jnp.float32)
        + benc_ref[...],
        0.0,
    )
    s = lax.dot_general(
        watt_ref[...], emb, (((1,), (1,)), ((), ())),
        preferred_element_type=jnp.float32,
    )
    out_ref[...] = s.reshape(1, 1, TILE_ROWS)


_scores_call = pl.pallas_call(
    _scores_body,
    grid=(TILES,),
    in_specs=[
        pl.BlockSpec((TILE_ROWS, F), lambda i: (i, 0)),
        pl.BlockSpec((F, Z), lambda i: (0, 0)),
        pl.BlockSpec((1, Z), lambda i: (0, 0)),
        pl.BlockSpec((1, Z), lambda i: (0, 0)),
    ],
    out_specs=pl.BlockSpec((1, 1, TILE_ROWS), lambda i: (i, 0, 0)),
    out_shape=jax.ShapeDtypeStruct((TILES, 1, TILE_ROWS), jnp.float32),
)


def _iota16():
    return lax.iota(jnp.int32, 16)


def _sc_body(scores_hbm, bags_hbm, rows_hbm, shist, scnt, sidx_hbm,
             kbuf, hist, allh, cbuf, allcnt, selbuf, sidxall, gidx16,
             rowbuf16, sem, sem2):
    wid = lax.axis_index("s")
    base = wid * CHUNK
    iota = _iota16()
    ones = jnp.ones((16,), jnp.int32)

    pltpu.sync_copy(scores_hbm.at[pl.ds(base, CHUNK)], kbuf)

    def key_body(i, c):
        ib = kbuf[pl.ds(i * 16, 16)]
        kbuf[pl.ds(i * 16, 16)] = jnp.where(ib < 0, ~ib, ib | MININT)
        return c

    lax.fori_loop(0, NV, key_body, 0)

    hi = jnp.int32(0)
    k_rem = jnp.int32(K)
    for shift in (24, 16, 8, 0):
        um = (0xFFFFFFFF << (shift + 8)) & 0xFFFFFFFF
        upper_mask = jnp.int32(um - 0x100000000 if um >= 0x80000000 else um)
        for v in range(16):
            hist[pl.ds(v * 16, 16)] = jnp.zeros((16,), jnp.int32)
        hi_upper = hi & upper_mask

        def hist_body(i, c, shift=shift, upper_mask=upper_mask, hi_upper=hi_upper):
            kb = kbuf[pl.ds(i * 16, 16)]
            byte = lax.shift_right_logical(kb, shift) & 0xFF
            match = (kb & upper_mask) == hi_upper
            plsc.addupdate_scatter(hist, [byte], ones, mask=match)
            return c

        lax.fori_loop(0, NV, hist_body, 0)

        pltpu.sync_copy(hist, shist.at[wid])
        plsc.subcore_barrier()
        pltpu.sync_copy(shist, allh)

        tot = []
        for v in range(16):
            tv = jnp.zeros((16,), jnp.int32)
            for t in range(16):
                tv = tv + allh[t, pl.ds(v * 16, 16)]
            tot.append(tv)
        desc_incl = [None] * 16
        carry = jnp.int32(0)
        for v in range(15, -1, -1):
            rv = lax.rev(tot[v], (0,))
            cs = plsc.cumsum(rv) + carry
            desc_incl[v] = lax.rev(cs, (0,))
            carry = carry + jnp.sum(tot[v])
        b_star = jnp.int32(0)
        new_krem = jnp.int32(0)
        for v in range(16):
            cgt = desc_incl[v] - tot[v]
            cond = (cgt < k_rem) & (desc_incl[v] >= k_rem)
            ids = iota + 16 * v
            b_star = b_star + jnp.sum(jnp.where(cond, ids, 0))
            new_krem = new_krem + jnp.sum(jnp.where(cond, k_rem - cgt, 0))
        k_rem = new_krem
        hi = hi | (b_star << shift)
        plsc.subcore_barrier()

    vk_bits = hi
    vk_s = vk_bits ^ MININT
    need = k_rem

    def cnt_body(i, c):
        cgt, ceq = c
        kb = kbuf[pl.ds(i * 16, 16)]
        sk = kb ^ MININT
        cgt = cgt + jnp.sum(jnp.where(sk > vk_s, 1, 0))
        ceq = ceq + jnp.sum(jnp.where(kb == vk_bits, 1, 0))
        return (cgt, ceq)

    cgt_me, ceq_me = lax.fori_loop(0, NV, cnt_body, (jnp.int32(0), jnp.int32(0)))

    cbuf[...] = jnp.where(iota == 0, cgt_me, jnp.where(iota == 1, ceq_me, 0))
    pltpu.sync_copy(cbuf, scnt.at[wid])
    plsc.subcore_barrier()
    pltpu.sync_copy(scnt, allcnt)

    my_pre_eq = jnp.int32(0)
    pre_eq = jnp.int32(0)
    pre_sel = jnp.int32(0)
    bases = []
    sizes = []
    for t in range(16):
        row = allcnt[t, :]
        g_t = jnp.sum(jnp.where(iota == 0, row, 0))
        e_t = jnp.sum(jnp.where(iota == 1, row, 0))
        take_t = jnp.clip(need - pre_eq, 0, e_t)
        is_me = jnp.int32(t) == wid
        my_pre_eq = jnp.where(is_me, pre_eq, my_pre_eq)
        bases.append(pre_sel)
        sizes.append(g_t + take_t)
        pre_eq = pre_eq + e_t
        pre_sel = pre_sel + g_t + take_t
    my_take = jnp.clip(need - my_pre_eq, 0, ceq_me)
    c_me = cgt_me + my_take

    for v in range(16):
        selbuf[pl.ds(v * 16, 16)] = jnp.zeros((16,), jnp.int32)

    def sel_body(i, c):
        sel_c, eq_c = c
        kb = kbuf[pl.ds(i * 16, 16)]
        sk = kb ^ MININT
        gtm = sk > vk_s
        eqm = kb == vk_bits
        eq_rank = plsc.cumsum(jnp.where(eqm, 1, 0)) - 1 + eq_c
        selm = gtm | (eqm & ((my_pre_eq + eq_rank) < need))
        seli = jnp.where(selm, 1, 0)
        pos = plsc.cumsum(seli) - 1 + sel_c
        gidx = base + i * 16 + iota
        plsc.store_scatter(selbuf, [pos], gidx, mask=selm)
        return (sel_c + jnp.sum(seli), eq_c + jnp.sum(jnp.where(eqm, 1, 0)))

    lax.fori_loop(0, NV, sel_body, (jnp.int32(0), jnp.int32(0)))

    pltpu.sync_copy(selbuf, sidx_hbm.at[wid])
    plsc.subcore_barrier()
    pltpu.sync_copy(sidx_hbm, sidxall)

    r = wid * 16 + iota
    src_t = jnp.zeros((16,), jnp.int32)
    src_o = jnp.zeros((16,), jnp.int32)
    for t in range(16):
        m = (r >= bases[t]) & (r < bases[t] + sizes[t])
        src_t = src_t + jnp.where(m, t, 0)
        src_o = src_o + jnp.where(m, r - bases[t], 0)
    gidx16[...] = plsc.load_gather(sidxall, [src_t, src_o])
    pltpu.async_copy(bags_hbm.at[gidx16], rowbuf16, sem).wait()
    pltpu.sync_copy(rowbuf16, rows_hbm.at[pl.ds(wid * 16, 16)])


_sc_mesh = plsc.VectorSubcoreMesh(
    core_axis_name="c", subcore_axis_name="s", num_cores=1, num_subcores=NSC)


def _make_sc_call(interpret=False):
    return pl.kernel(
        _sc_body,
        out_type=[
            jax.ShapeDtypeStruct((K, F), jnp.float32),
            jax.ShapeDtypeStruct((16, 256), jnp.int32),
            jax.ShapeDtypeStruct((16, 16), jnp.int32),
            jax.ShapeDtypeStruct((16, K), jnp.int32),
        ],
        mesh=_sc_mesh,
        scratch_types=[
            pltpu.VMEM((CHUNK,), jnp.int32),
            pltpu.VMEM((256,), jnp.int32),
            pltpu.VMEM((16, 256), jnp.int32),
            pltpu.VMEM((16,), jnp.int32),
            pltpu.VMEM((16, 16), jnp.int32),
            pltpu.VMEM((K,), jnp.int32),
            pltpu.VMEM((16, K), jnp.int32),
            pltpu.VMEM((16,), jnp.int32),
            pltpu.VMEM((16, F), jnp.float32),
            pltpu.SemaphoreType.DMA,
            pltpu.SemaphoreType.DMA,
        ],
        compiler_params=pltpu.CompilerParams(needs_layout_passes=False),
        interpret=interpret,
    )


_sc_call = _make_sc_call()


def _head_body(rows_ref, wenc_ref, benc_ref, gamma_ref, beta_ref,
               mean_ref, var_ref, whead_ref, bhead_ref, out_ref):
    emb = jnp.maximum(
        jnp.dot(rows_ref[...], wenc_ref[...], preferred_element_type=jnp.float32)
        + benc_ref[...],
        0.0,
    )
    pooled = jnp.sum(emb, axis=0, keepdims=True) * (1.0 / K)
    h = (pooled - mean_ref[...]) * lax.rsqrt(var_ref[...] + 1e-5) * gamma_ref[...] + beta_ref[...]
    out_ref[...] = jnp.dot(h, whead_ref[...], preferred_element_type=jnp.float32) + bhead_ref[...]


_head_call = pl.pallas_call(
    _head_body,
    in_specs=[
        pl.BlockSpec((K, F), lambda: (0, 0)),
        pl.BlockSpec((F, Z), lambda: (0, 0)),
        pl.BlockSpec((1, Z), lambda: (0, 0)),
        pl.BlockSpec((1, Z), lambda: (0, 0)),
        pl.BlockSpec((1, Z), lambda: (0, 0)),
        pl.BlockSpec((1, Z), lambda: (0, 0)),
        pl.BlockSpec((1, Z), lambda: (0, 0)),
        pl.BlockSpec((Z, 2), lambda: (0, 0)),
        pl.BlockSpec((1, 2), lambda: (0, 0)),
    ],
    out_specs=pl.BlockSpec((1, 2), lambda: (0, 0)),
    out_shape=jax.ShapeDtypeStruct((1, 2), jnp.float32),
)


def kernel(bags, W_enc, b_enc, W_att, b_att, bn_gamma, bn_beta, bn_mean,
           bn_var, W_head, b_head):
    benc2 = b_enc.reshape(1, Z)
    watt2 = W_att.reshape(1, Z)
    scores = _scores_call(bags, W_enc, benc2, watt2).reshape(N)
    scores_i32 = lax.bitcast_convert_type(scores, jnp.int32)
    rows = _sc_call(scores_i32, bags)[0]
    out = _head_call(
        rows, W_enc, benc2,
        bn_gamma.reshape(1, Z), bn_beta.reshape(1, Z),
        bn_mean.reshape(1, Z), bn_var.reshape(1, Z),
        W_head, b_head.reshape(1, 2),
    )
    return out.reshape(2)
